# baseline (speedup 1.0000x reference)
import functools

import jax
import jax.numpy as jnp
from jax import lax
from jax.experimental import pallas as pl
from jax.experimental.pallas import tpu as pltpu
from jax.experimental.pallas import tpu_sc as plsc

N = 10000
EM = 160000
EW = 32000
LAT = 128

NC = 2
NS = 16
NW = NC * NS

GCH = 200
SCH = 192
EM_W = EM // NW
EW_W = EW // NW
EM_C = EM // NC
EM_T = EM_C // NS
EW_C = EW // NC
EW_T = EW_C // NS
NCH_M = EM_T // SCH
TAIL_M = EM_T - NCH_M * SCH
NCH_W = EW_T // SCH
TAIL_W = EW_T - NCH_W * SCH
NROW = 624
NTAIL = N - NROW * NS

def _make_gather_body(ept):
    nch = ept // GCH
    assert nch % 2 == 1

    def body(tabs, tabr, sidx, ridx, outg,
             idxs, idxr, bufs0, bufs1, sem0, sem1):
        c = lax.axis_index("c")
        s = lax.axis_index("s")
        base = (s * NC + c) * ept
        pltpu.sync_copy(sidx.at[pl.ds(base, ept)], idxs)
        pltpu.sync_copy(ridx.at[pl.ds(base, ept)], idxr)

        def fire(off, bs, sem):
            pltpu.async_copy(tabs.at[idxs.at[pl.ds(off, GCH)]], bs, sem)

        def drain(off, bs, sem):
            pltpu.make_async_copy(tabs.at[idxs.at[pl.ds(off, GCH)]], bs, sem).wait()
            pltpu.async_copy(tabr.at[idxr.at[pl.ds(off, GCH)]], bs, sem, add=True)
            pltpu.make_async_copy(tabr.at[idxr.at[pl.ds(off, GCH)]], bs, sem).wait()
            pltpu.sync_copy(bs, outg.at[pl.ds(base + off, GCH)])

        fire(0, bufs0, sem0)

        def pair(j, carry):
            o0 = (2 * j) * GCH
            fire(o0 + GCH, bufs1, sem1)
            drain(o0, bufs0, sem0)
            fire(o0 + 2 * GCH, bufs0, sem0)
            drain(o0 + GCH, bufs1, sem1)
            return carry

        lax.fori_loop(0, (nch - 1) // 2, pair, 0)
        drain((nch - 1) * GCH, bufs0, sem0)

    return body


def _make_scatter_body(ept, nch, tail):
    def body(src, ridx, zz, out, acc, vals0, vals1, idxb0, idxb1, idxt,
             lsem0, lsem1):
        c = lax.axis_index("c")
        s = lax.axis_index("s")
        rbase = s * NROW
        tb = NROW * NS
        last = s == NS - 1
        tbase0 = (c * NS + s) * ept

        pltpu.sync_copy(zz, acc.at[pl.ds(rbase, NROW)])

        @pl.when(last)
        def _():
            pltpu.sync_copy(zz.at[pl.ds(0, NTAIL)], acc.at[pl.ds(tb, NTAIL)])

        plsc.subcore_barrier()

        def fire(off, vb, ib, sem):
            pltpu.async_copy(src.at[pl.ds(tbase0 + off, SCH)], vb, sem)
            pltpu.async_copy(ridx.at[pl.ds(tbase0 + off, SCH)], ib, sem)

        def drain(off, vb, ib, sem):
            pltpu.make_async_copy(src.at[pl.ds(tbase0 + off, SCH)], vb, sem).wait()
            pltpu.make_async_copy(ridx.at[pl.ds(tbase0 + off, SCH)], ib, sem).wait()
            pltpu.sync_copy(vb, acc.at[ib], add=True)

        fire(0, vals0, idxb0, lsem0)

        def pair(j, carry):
            o0 = (2 * j) * SCH
            fire(o0 + SCH, vals1, idxb1, lsem1)
            drain(o0, vals0, idxb0, lsem0)
            fire(o0 + 2 * SCH, vals0, idxb0, lsem0)
            drain(o0 + SCH, vals1, idxb1, lsem1)
            return carry

        lax.fori_loop(0, (nch - 1) // 2, pair, 0)
        if nch % 2:
            drain((nch - 1) * SCH, vals0, idxb0, lsem0)
        else:
            drain((nch - 2) * SCH, vals0, idxb0, lsem0)
            fire((nch - 1) * SCH, vals1, idxb1, lsem1)
            drain((nch - 1) * SCH, vals1, idxb1, lsem1)
        toff = tbase0 + nch * SCH
        pltpu.sync_copy(ridx.at[pl.ds(toff, tail)], idxt)
        pltpu.sync_copy(src.at[pl.ds(toff, tail)], vals0.at[pl.ds(0, tail)])
        pltpu.sync_copy(vals0.at[pl.ds(0, tail)], acc.at[idxt], add=True)

        plsc.subcore_barrier()
        pltpu.sync_copy(acc.at[pl.ds(rbase, NROW)],
                        out.at[pl.ds(c * N + rbase, NROW)])

        @pl.when(last)
        def _():
            pltpu.sync_copy(acc.at[pl.ds(tb, NTAIL)],
                            out.at[pl.ds(c * N + tb, NTAIL)])

    return body


@functools.cache
def _sc_gather(e):
    mesh = plsc.VectorSubcoreMesh(core_axis_name="c", subcore_axis_name="s")
    ept = e // NW
    return pl.kernel(
        _make_gather_body(ept),
        out_type=jax.ShapeDtypeStruct((e, LAT), jnp.float32),
        mesh=mesh,
        scratch_types=[
            pltpu.VMEM((ept,), jnp.int32),
            pltpu.VMEM((ept,), jnp.int32),
            pltpu.VMEM((GCH, LAT), jnp.float32),
            pltpu.VMEM((GCH, LAT), jnp.float32),
            pltpu.SemaphoreType.DMA,
            pltpu.SemaphoreType.DMA,
        ],
    )


@functools.cache
def _sc_scatter(e):
    mesh = plsc.VectorSubcoreMesh(core_axis_name="c", subcore_axis_name="s")
    ept = e // NW
    nch = ept // SCH
    tail = ept - nch * SCH
    return pl.kernel(
        _make_scatter_body(ept, nch, tail),
        out_type=jax.ShapeDtypeStruct((NC * N, LAT), jnp.float32),
        mesh=mesh,
        scratch_types=[
            pltpu.VMEM_SHARED((N, LAT), jnp.float32),
            pltpu.VMEM((SCH, LAT), jnp.float32),
            pltpu.VMEM((SCH, LAT), jnp.float32),
            pltpu.VMEM((SCH,), jnp.int32),
            pltpu.VMEM((SCH,), jnp.int32),
            pltpu.VMEM((tail,), jnp.int32),
            pltpu.SemaphoreType.DMA,
            pltpu.SemaphoreType.DMA,
        ],
    )


def _ln(h, g, be):
    mu = jnp.mean(h, axis=-1, keepdims=True)
    var = jnp.mean((h - mu) ** 2, axis=-1, keepdims=True)
    h = (h - mu) * lax.rsqrt(var + 1e-5)
    return h * g + be


def _dot(a, b):
    return jnp.dot(a, b, precision=lax.Precision.DEFAULT,
                   preferred_element_type=jnp.float32)


def _mlp_tail(h, w2, b2, w3, b3, w4, b4, g, be):
    h = jnp.maximum(h, 0.0)
    h = jnp.maximum(_dot(h, w2[...]) + b2[...], 0.0)
    h = jnp.maximum(_dot(h, w3[...]) + b3[...], 0.0)
    h = _dot(h, w4[...]) + b4[...]
    return _ln(h, g[...], be[...])


def _enc_body(x, w1, b1, w2, b2, w3, b3, w4, b4, g, be, o_ref):
    h = _dot(x[...], w1[...]) + b1[...]
    o_ref[...] = _mlp_tail(h, w2, b2, w3, b3, w4, b4, g, be)


def _proj_body(nl, wam, wbm, b1m, waw, wbw, b1w,
               psm_ref, prm_ref, psw_ref, prw_ref):
    x = nl[...]
    psm_ref[...] = _dot(x, wam[...]) + b1m[...]
    prm_ref[...] = _dot(x, wbm[...])
    psw_ref[...] = _dot(x, waw[...]) + b1w[...]
    prw_ref[...] = _dot(x, wbw[...])


def _edge_body(gsum, el, wc, w2, b2, w3, b3, w4, b4, g, be, up_ref):
    x = el[...]
    h = gsum[...] + _dot(x, wc[...])
    ne = _mlp_tail(h, w2, b2, w3, b3, w4, b4, g, be)
    up_ref[...] = x + ne


def _node_body(nl, pm0, pm1, qm0, qm1, pw0, pw1, qw0, qw1,
               wa, wb, wc, b1, w2, b2, w3, b3, w4, b4, g, be, o_ref):
    x = nl[...]
    am = pm0[...] + pm1[...] - qm0[...] - qm1[...]
    aw = pw0[...] + pw1[...] - qw0[...] - qw1[...]
    h = (_dot(x, wa[...]) + _dot(am, wb[...])
         + _dot(aw, wc[...]) + b1[...])
    o_ref[...] = x + _mlp_tail(h, w2, b2, w3, b3, w4, b4, g, be)


def _full(a):
    nd = a.ndim
    return pl.BlockSpec(a.shape, lambda i, _nd=nd: (0,) * _nd)


def _rows(blk, width):
    return pl.BlockSpec((blk, width), lambda i: (i, 0))


def _run_enc(x, p, blk):
    e, ind = x.shape
    ws = list(p["W"])
    bs = [b.reshape(1, LAT) for b in p["b"]]
    g = p["g"].reshape(1, LAT)
    be = p["be"].reshape(1, LAT)
    flat = [ws[0], bs[0], ws[1], bs[1], ws[2], bs[2], ws[3], bs[3], g, be]
    return pl.pallas_call(
        _enc_body,
        grid=(e // blk,),
        in_specs=[_rows(blk, ind)] + [_full(a) for a in flat],
        out_specs=_rows(blk, LAT),
        out_shape=jax.ShapeDtypeStruct((e, LAT), jnp.float32),
    )(x, *flat)


def _run_proj(nl, wm, ww, blk):
    flat = [wm["wa"], wm["wb"], wm["b1"], ww["wa"], ww["wb"], ww["b1"]]
    return pl.pallas_call(
        _proj_body,
        grid=(N // blk,),
        in_specs=[_rows(blk, LAT)] + [_full(a) for a in flat],
        out_specs=[_rows(blk, LAT)] * 4,
        out_shape=[jax.ShapeDtypeStruct((N, LAT), jnp.float32)] * 4,
    )(nl, *flat)


def _run_edge(gsum, el, w, blk):
    e = el.shape[0]
    flat = [w["wc"], w["w2"], w["b2"], w["w3"],
            w["b3"], w["w4"], w["b4"], w["g"], w["be"]]
    return pl.pallas_call(
        _edge_body,
        grid=(e // blk,),
        in_specs=[_rows(blk, LAT)] * 2 + [_full(a) for a in flat],
        out_specs=_rows(blk, LAT),
        out_shape=jax.ShapeDtypeStruct((e, LAT), jnp.float32),
    )(gsum, el, *flat)


def _run_node(nl, pm, qm, pw, qw, w, blk):
    flat = [w["wa"], w["wb"], w["wc"], w["b1"], w["w2"], w["b2"], w["w3"],
            w["b3"], w["w4"], w["b4"], w["g"], w["be"]]
    nb = N // blk
    part0 = pl.BlockSpec((blk, LAT), lambda i: (i, 0))
    part1 = pl.BlockSpec((blk, LAT), lambda i, _nb=nb: (i + _nb, 0))
    return pl.pallas_call(
        _node_body,
        grid=(nb,),
        in_specs=[_rows(blk, LAT), part0, part1, part0, part1,
                  part0, part1, part0, part1]
        + [_full(a) for a in flat],
        out_specs=_rows(blk, LAT),
        out_shape=jax.ShapeDtypeStruct((N, LAT), jnp.float32),
    )(nl, pm, pm, qm, qm, pw, pw, qw, qw, *flat)


def _stack_block(blocks, role):
    def cat(f):
        return jnp.stack([f(b[role]) for b in blocks])

    return {
        "wa": cat(lambda p: p["W"][0][:LAT]),
        "wb": cat(lambda p: p["W"][0][LAT:2 * LAT]),
        "wc": cat(lambda p: p["W"][0][2 * LAT:]),
        "b1": cat(lambda p: p["b"][0].reshape(1, LAT)),
        "w2": cat(lambda p: p["W"][1]),
        "b2": cat(lambda p: p["b"][1].reshape(1, LAT)),
        "w3": cat(lambda p: p["W"][2]),
        "b3": cat(lambda p: p["b"][2].reshape(1, LAT)),
        "w4": cat(lambda p: p["W"][3]),
        "b4": cat(lambda p: p["b"][3].reshape(1, LAT)),
        "g": cat(lambda p: p["g"].reshape(1, LAT)),
        "be": cat(lambda p: p["be"].reshape(1, LAT)),
    }


def kernel(node_features, mesh_features, world_features, params,
           mesh_senders, mesh_receivers, world_senders, world_receivers):
    ms = mesh_senders.astype(jnp.int32)
    mr = mesh_receivers.astype(jnp.int32)
    ws = world_senders.astype(jnp.int32)
    wr = world_receivers.astype(jnp.int32)

    enc = params["enc"]
    node_lat = _run_enc(node_features, enc["node"], 2000)
    mesh_lat = _run_enc(mesh_features, enc["mesh"], 8000)
    world_lat = _run_enc(world_features, enc["world"], 8000)

    blocks = params["blocks"]
    wts = {
        "mesh": _stack_block(blocks, "mesh"),
        "world": _stack_block(blocks, "world"),
        "node": _stack_block(blocks, "node"),
    }
    zz = jnp.zeros((NROW, LAT), jnp.float32)

    pm0 = _sc_scatter(EM)(mesh_lat, mr, zz)
    pw0 = _sc_scatter(EW)(world_lat, wr, zz)

    def step(carry, w):
        nl, ml, wl, qm, qw = carry
        psm, prm, psw, prw = _run_proj(nl, w["mesh"], w["world"], 2000)
        gm = _sc_gather(EM)(psm, prm, ms, mr)
        gw = _sc_gather(EW)(psw, prw, ws, wr)
        ml2 = _run_edge(gm, ml, w["mesh"], 8000)
        pm = _sc_scatter(EM)(ml2, mr, zz)
        wl2 = _run_edge(gw, wl, w["world"], 8000)
        pw = _sc_scatter(EW)(wl2, wr, zz)
        nl2 = _run_node(nl, pm, qm, pw, qw, w["node"], 2000)
        return (nl2, ml2, wl2, pm, pw), None

    (node_lat, mesh_lat, world_lat, _, _), _ = lax.scan(
        step, (node_lat, mesh_lat, world_lat, pm0, pw0), wts)
    return (node_lat, mesh_lat, world_lat)


# speedup vs baseline: 1.2576x; 1.2576x over previous
import functools

import jax
import jax.numpy as jnp
from jax import lax
from jax.experimental import pallas as pl
from jax.experimental.pallas import tpu as pltpu
from jax.experimental.pallas import tpu_sc as plsc

N = 10000
EM = 160000
EW = 32000
LAT = 128

NC = 2
NS = 16
NW = NC * NS

GCH = 200
SCH = 192
EM_W = EM // NW
EW_W = EW // NW
EM_C = EM // NC
EM_T = EM_C // NS
EW_C = EW // NC
EW_T = EW_C // NS
NCH_M = EM_T // SCH
TAIL_M = EM_T - NCH_M * SCH
NCH_W = EW_T // SCH
TAIL_W = EW_T - NCH_W * SCH
NROW = 624
NTAIL = N - NROW * NS

def _make_gather_body(ept):
    nch = ept // GCH
    assert nch % 2 == 1

    def body(tabs, tabr, sidx, ridx, outg,
             idxs, idxr, bufs0, bufs1, sem0, sem1):
        c = lax.axis_index("c")
        s = lax.axis_index("s")
        base = (s * NC + c) * ept
        pltpu.sync_copy(sidx.at[pl.ds(base, ept)], idxs)
        pltpu.sync_copy(ridx.at[pl.ds(base, ept)], idxr)

        def fire(off, bs, sem):
            pltpu.async_copy(tabs.at[idxs.at[pl.ds(off, GCH)]], bs, sem)

        def drain(off, bs, sem):
            pltpu.make_async_copy(tabs.at[idxs.at[pl.ds(off, GCH)]], bs, sem).wait()
            pltpu.async_copy(tabr.at[idxr.at[pl.ds(off, GCH)]], bs, sem, add=True)
            pltpu.make_async_copy(tabr.at[idxr.at[pl.ds(off, GCH)]], bs, sem).wait()
            pltpu.sync_copy(bs, outg.at[pl.ds(base + off, GCH)])

        fire(0, bufs0, sem0)

        def pair(j, carry):
            o0 = (2 * j) * GCH
            fire(o0 + GCH, bufs1, sem1)
            drain(o0, bufs0, sem0)
            fire(o0 + 2 * GCH, bufs0, sem0)
            drain(o0 + GCH, bufs1, sem1)
            return carry

        lax.fori_loop(0, (nch - 1) // 2, pair, 0)
        drain((nch - 1) * GCH, bufs0, sem0)

    return body


def _make_scatter_body(ept, nch, tail):
    def body(src, ridx, zz, out, acc, vals0, vals1, idxb0, idxb1, idxt,
             lsem0, lsem1):
        c = lax.axis_index("c")
        s = lax.axis_index("s")
        rbase = s * NROW
        tb = NROW * NS
        last = s == NS - 1
        tbase0 = (c * NS + s) * ept

        pltpu.sync_copy(zz, acc.at[pl.ds(rbase, NROW)])

        @pl.when(last)
        def _():
            pltpu.sync_copy(zz.at[pl.ds(0, NTAIL)], acc.at[pl.ds(tb, NTAIL)])

        plsc.subcore_barrier()

        def fire(off, vb, ib, sem):
            pltpu.async_copy(src.at[pl.ds(tbase0 + off, SCH)], vb, sem)
            pltpu.async_copy(ridx.at[pl.ds(tbase0 + off, SCH)], ib, sem)

        def drain(off, vb, ib, sem):
            pltpu.make_async_copy(src.at[pl.ds(tbase0 + off, SCH)], vb, sem).wait()
            pltpu.make_async_copy(ridx.at[pl.ds(tbase0 + off, SCH)], ib, sem).wait()
            pltpu.sync_copy(vb, acc.at[ib], add=True)

        fire(0, vals0, idxb0, lsem0)

        def pair(j, carry):
            o0 = (2 * j) * SCH
            fire(o0 + SCH, vals1, idxb1, lsem1)
            drain(o0, vals0, idxb0, lsem0)
            fire(o0 + 2 * SCH, vals0, idxb0, lsem0)
            drain(o0 + SCH, vals1, idxb1, lsem1)
            return carry

        lax.fori_loop(0, (nch - 1) // 2, pair, 0)
        if nch % 2:
            drain((nch - 1) * SCH, vals0, idxb0, lsem0)
        else:
            drain((nch - 2) * SCH, vals0, idxb0, lsem0)
            fire((nch - 1) * SCH, vals1, idxb1, lsem1)
            drain((nch - 1) * SCH, vals1, idxb1, lsem1)
        toff = tbase0 + nch * SCH
        pltpu.sync_copy(ridx.at[pl.ds(toff, tail)], idxt)
        pltpu.sync_copy(src.at[pl.ds(toff, tail)], vals0.at[pl.ds(0, tail)])
        pltpu.sync_copy(vals0.at[pl.ds(0, tail)], acc.at[idxt], add=True)

        plsc.subcore_barrier()
        pltpu.sync_copy(acc.at[pl.ds(rbase, NROW)],
                        out.at[pl.ds(c * N + rbase, NROW)])

        @pl.when(last)
        def _():
            pltpu.sync_copy(acc.at[pl.ds(tb, NTAIL)],
                            out.at[pl.ds(c * N + tb, NTAIL)])

    return body


@functools.cache
def _sc_gather(e):
    mesh = plsc.VectorSubcoreMesh(core_axis_name="c", subcore_axis_name="s")
    ept = e // NW
    return pl.kernel(
        _make_gather_body(ept),
        out_type=jax.ShapeDtypeStruct((e, LAT), jnp.float32),
        mesh=mesh,
        scratch_types=[
            pltpu.VMEM((ept,), jnp.int32),
            pltpu.VMEM((ept,), jnp.int32),
            pltpu.VMEM((GCH, LAT), jnp.float32),
            pltpu.VMEM((GCH, LAT), jnp.float32),
            pltpu.SemaphoreType.DMA,
            pltpu.SemaphoreType.DMA,
        ],
    )


@functools.cache
def _sc_scatter(e):
    mesh = plsc.VectorSubcoreMesh(core_axis_name="c", subcore_axis_name="s")
    ept = e // NW
    nch = ept // SCH
    tail = ept - nch * SCH
    return pl.kernel(
        _make_scatter_body(ept, nch, tail),
        out_type=jax.ShapeDtypeStruct((NC * N, LAT), jnp.float32),
        mesh=mesh,
        scratch_types=[
            pltpu.VMEM_SHARED((N, LAT), jnp.float32),
            pltpu.VMEM((SCH, LAT), jnp.float32),
            pltpu.VMEM((SCH, LAT), jnp.float32),
            pltpu.VMEM((SCH,), jnp.int32),
            pltpu.VMEM((SCH,), jnp.int32),
            pltpu.VMEM((tail,), jnp.int32),
            pltpu.SemaphoreType.DMA,
            pltpu.SemaphoreType.DMA,
        ],
    )


def _ln(h, g, be):
    mu = jnp.mean(h, axis=-1, keepdims=True)
    var = jnp.mean((h - mu) ** 2, axis=-1, keepdims=True)
    h = (h - mu) * lax.rsqrt(var + 1e-5)
    return h * g + be


def _dot(a, b):
    return jnp.dot(a, b, precision=lax.Precision.DEFAULT,
                   preferred_element_type=jnp.float32)


def _mlp_tail(h, w2, b2, w3, b3, w4, b4, g, be):
    h = jnp.maximum(h, 0.0)
    h = jnp.maximum(_dot(h, w2[...]) + b2[...], 0.0)
    h = jnp.maximum(_dot(h, w3[...]) + b3[...], 0.0)
    h = _dot(h, w4[...]) + b4[...]
    return _ln(h, g[...], be[...])


def _enc_body(x, w1, b1, w2, b2, w3, b3, w4, b4, g, be, o_ref):
    h = _dot(x[...], w1[...]) + b1[...]
    o_ref[...] = _mlp_tail(h, w2, b2, w3, b3, w4, b4, g, be)


def _proj_body(nl, wam, wbm, b1m, waw, wbw, b1w,
               psm_ref, prm_ref, psw_ref, prw_ref):
    x = nl[...]
    psm_ref[...] = _dot(x, wam[...]) + b1m[...]
    prm_ref[...] = _dot(x, wbm[...])
    psw_ref[...] = _dot(x, waw[...]) + b1w[...]
    prw_ref[...] = _dot(x, wbw[...])


def _edge_body(gsum, el, wc, w2, b2, w3, b3, w4, b4, g, be, up_ref):
    x = el[...]
    h = gsum[...] + _dot(x, wc[...])
    ne = _mlp_tail(h, w2, b2, w3, b3, w4, b4, g, be)
    up_ref[...] = x + ne


def _node_body(nl, pm0, pm1, qm0, qm1, pw0, pw1, qw0, qw1,
               wa, wb, wc, b1, w2, b2, w3, b3, w4, b4, g, be, o_ref):
    x = nl[...]
    am = pm0[...] + pm1[...] - qm0[...] - qm1[...]
    aw = pw0[...] + pw1[...] - qw0[...] - qw1[...]
    h = (_dot(x, wa[...]) + _dot(am, wb[...])
         + _dot(aw, wc[...]) + b1[...])
    o_ref[...] = x + _mlp_tail(h, w2, b2, w3, b3, w4, b4, g, be)


def _full(a):
    nd = a.ndim
    return pl.BlockSpec(a.shape, lambda i, _nd=nd: (0,) * _nd)


def _rows(blk, width):
    return pl.BlockSpec((blk, width), lambda i: (i, 0))


def _run_enc(x, p, blk):
    e, ind = x.shape
    ws = list(p["W"])
    bs = [b.reshape(1, LAT) for b in p["b"]]
    g = p["g"].reshape(1, LAT)
    be = p["be"].reshape(1, LAT)
    flat = [ws[0], bs[0], ws[1], bs[1], ws[2], bs[2], ws[3], bs[3], g, be]
    return pl.pallas_call(
        _enc_body,
        grid=(e // blk,),
        in_specs=[_rows(blk, ind)] + [_full(a) for a in flat],
        out_specs=_rows(blk, LAT),
        out_shape=jax.ShapeDtypeStruct((e, LAT), jnp.float32),
    )(x, *flat)


def _run_proj(nl, wm, ww, blk):
    flat = [wm["wa"], wm["wb"], wm["b1"], ww["wa"], ww["wb"], ww["b1"]]
    return pl.pallas_call(
        _proj_body,
        grid=(N // blk,),
        in_specs=[_rows(blk, LAT)] + [_full(a) for a in flat],
        out_specs=[_rows(blk, LAT)] * 4,
        out_shape=[jax.ShapeDtypeStruct((N, LAT), jnp.float32)] * 4,
    )(nl, *flat)


def _run_edge(gsum, el, w, blk):
    e = el.shape[0]
    flat = [w["wc"], w["w2"], w["b2"], w["w3"],
            w["b3"], w["w4"], w["b4"], w["g"], w["be"]]
    return pl.pallas_call(
        _edge_body,
        grid=(e // blk,),
        in_specs=[_rows(blk, LAT)] * 2 + [_full(a) for a in flat],
        out_specs=_rows(blk, LAT),
        out_shape=jax.ShapeDtypeStruct((e, LAT), jnp.float32),
    )(gsum, el, *flat)


def _run_node(nl, pm, qm, pw, qw, w, blk):
    flat = [w["wa"], w["wb"], w["wc"], w["b1"], w["w2"], w["b2"], w["w3"],
            w["b3"], w["w4"], w["b4"], w["g"], w["be"]]
    nb = N // blk
    part0 = pl.BlockSpec((blk, LAT), lambda i: (i, 0))
    part1 = pl.BlockSpec((blk, LAT), lambda i, _nb=nb: (i + _nb, 0))
    return pl.pallas_call(
        _node_body,
        grid=(nb,),
        in_specs=[_rows(blk, LAT), part0, part1, part0, part1,
                  part0, part1, part0, part1]
        + [_full(a) for a in flat],
        out_specs=_rows(blk, LAT),
        out_shape=jax.ShapeDtypeStruct((N, LAT), jnp.float32),
    )(nl, pm, pm, qm, qm, pw, pw, qw, qw, *flat)


def _stack_block(blocks, role):
    def cat(f):
        return jnp.stack([f(b[role]) for b in blocks])

    return {
        "wa": cat(lambda p: p["W"][0][:LAT]),
        "wb": cat(lambda p: p["W"][0][LAT:2 * LAT]),
        "wc": cat(lambda p: p["W"][0][2 * LAT:]),
        "b1": cat(lambda p: p["b"][0].reshape(1, LAT)),
        "w2": cat(lambda p: p["W"][1]),
        "b2": cat(lambda p: p["b"][1].reshape(1, LAT)),
        "w3": cat(lambda p: p["W"][2]),
        "b3": cat(lambda p: p["b"][2].reshape(1, LAT)),
        "w4": cat(lambda p: p["W"][3]),
        "b4": cat(lambda p: p["b"][3].reshape(1, LAT)),
        "g": cat(lambda p: p["g"].reshape(1, LAT)),
        "be": cat(lambda p: p["be"].reshape(1, LAT)),
    }


def kernel(node_features, mesh_features, world_features, params,
           mesh_senders, mesh_receivers, world_senders, world_receivers):
    ms = mesh_senders.astype(jnp.int32)
    mr = mesh_receivers.astype(jnp.int32)
    ws = world_senders.astype(jnp.int32)
    wr = world_receivers.astype(jnp.int32)

    enc = params["enc"]
    node_lat = _run_enc(node_features, enc["node"], 2000)
    mesh_lat = _run_enc(mesh_features, enc["mesh"], 8000)
    world_lat = _run_enc(world_features, enc["world"], 8000)

    blocks = params["blocks"]
    wts = {
        "mesh": _stack_block(blocks, "mesh"),
        "world": _stack_block(blocks, "world"),
        "node": _stack_block(blocks, "node"),
    }
    zz = jnp.zeros((NROW, LAT), jnp.float32)

    pm0 = _sc_scatter(EM)(mesh_lat, mr, zz)
    pw0 = _sc_scatter(EW)(world_lat, wr, zz)

    def step(carry, w):
        nl, ml, wl, qm, qw = carry
        psm, prm, psw, prw = _run_proj(nl, w["mesh"], w["world"], 2000)
        gm = _sc_gather(EM)(psm, prm, ms, mr)
        gw = _sc_gather(EW)(psw, prw, ws, wr)
        ml2 = _run_edge(gm, ml, w["mesh"], 8000)
        pm = _sc_scatter(EM)(ml2, mr, zz)
        wl2 = _run_edge(gw, wl, w["world"], 8000)
        pw = _sc_scatter(EW)(wl2, wr, zz)
        nl2 = _run_node(nl, pm, qm, pw, qw, w["node"], 2000)
        return (nl2, ml2, wl2, pm, pw), None

    (node_lat, mesh_lat, world_lat, _, _), _ = lax.scan(
        step, (node_lat, mesh_lat, world_lat, pm0, pw0), wts, unroll=18)
    return (node_lat, mesh_lat, world_lat)


# speedup vs baseline: 1.2854x; 1.0221x over previous
import functools

import jax
import jax.numpy as jnp
from jax import lax
from jax.experimental import pallas as pl
from jax.experimental.pallas import tpu as pltpu
from jax.experimental.pallas import tpu_sc as plsc

N = 10000
EM = 160000
EW = 32000
LAT = 128

NC = 2
NS = 16
NW = NC * NS

GCH = 200
SCH = 192
EM_W = EM // NW
EW_W = EW // NW
EM_C = EM // NC
EM_T = EM_C // NS
EW_C = EW // NC
EW_T = EW_C // NS
NCH_M = EM_T // SCH
TAIL_M = EM_T - NCH_M * SCH
NCH_W = EW_T // SCH
TAIL_W = EW_T - NCH_W * SCH
NROW = 624
NTAIL = N - NROW * NS

def _make_gather_body(ept):
    nch = ept // GCH
    assert nch % 2 == 1

    def body(tabs, tabr, sidx, ridx, outg,
             idxs, idxr, bufs0, bufs1, sem0, sem1):
        c = lax.axis_index("c")
        s = lax.axis_index("s")
        base = (s * NC + c) * ept
        pltpu.sync_copy(sidx.at[pl.ds(base, ept)], idxs)
        pltpu.sync_copy(ridx.at[pl.ds(base, ept)], idxr)

        def fire(off, bs, sem):
            pltpu.async_copy(tabs.at[idxs.at[pl.ds(off, GCH)]], bs, sem)

        def drain(off, bs, sem):
            pltpu.make_async_copy(tabs.at[idxs.at[pl.ds(off, GCH)]], bs, sem).wait()
            pltpu.async_copy(tabr.at[idxr.at[pl.ds(off, GCH)]], bs, sem, add=True)
            pltpu.make_async_copy(tabr.at[idxr.at[pl.ds(off, GCH)]], bs, sem).wait()
            pltpu.sync_copy(bs, outg.at[pl.ds(base + off, GCH)])

        fire(0, bufs0, sem0)

        def pair(j, carry):
            o0 = (2 * j) * GCH
            fire(o0 + GCH, bufs1, sem1)
            drain(o0, bufs0, sem0)
            fire(o0 + 2 * GCH, bufs0, sem0)
            drain(o0 + GCH, bufs1, sem1)
            return carry

        lax.fori_loop(0, (nch - 1) // 2, pair, 0)
        drain((nch - 1) * GCH, bufs0, sem0)

    return body


def _make_scatter_body(ept, nch, tail):
    def body(src, ridx, zz, out, acc, vals0, vals1, idxb0, idxb1, idxt,
             lsem0, lsem1):
        c = lax.axis_index("c")
        s = lax.axis_index("s")
        rbase = s * NROW
        tb = NROW * NS
        last = s == NS - 1
        tbase0 = (c * NS + s) * ept

        pltpu.sync_copy(zz, acc.at[pl.ds(rbase, NROW)])

        @pl.when(last)
        def _():
            pltpu.sync_copy(zz.at[pl.ds(0, NTAIL)], acc.at[pl.ds(tb, NTAIL)])

        plsc.subcore_barrier()

        def fire(off, vb, ib, sem):
            pltpu.async_copy(src.at[pl.ds(tbase0 + off, SCH)], vb, sem)
            pltpu.async_copy(ridx.at[pl.ds(tbase0 + off, SCH)], ib, sem)

        def drain(off, vb, ib, sem):
            pltpu.make_async_copy(src.at[pl.ds(tbase0 + off, SCH)], vb, sem).wait()
            pltpu.make_async_copy(ridx.at[pl.ds(tbase0 + off, SCH)], ib, sem).wait()
            pltpu.sync_copy(vb, acc.at[ib], add=True)

        fire(0, vals0, idxb0, lsem0)

        def pair(j, carry):
            o0 = (2 * j) * SCH
            fire(o0 + SCH, vals1, idxb1, lsem1)
            drain(o0, vals0, idxb0, lsem0)
            fire(o0 + 2 * SCH, vals0, idxb0, lsem0)
            drain(o0 + SCH, vals1, idxb1, lsem1)
            return carry

        lax.fori_loop(0, (nch - 1) // 2, pair, 0)
        if nch % 2:
            drain((nch - 1) * SCH, vals0, idxb0, lsem0)
        else:
            drain((nch - 2) * SCH, vals0, idxb0, lsem0)
            fire((nch - 1) * SCH, vals1, idxb1, lsem1)
            drain((nch - 1) * SCH, vals1, idxb1, lsem1)
        toff = tbase0 + nch * SCH
        pltpu.sync_copy(ridx.at[pl.ds(toff, tail)], idxt)
        pltpu.sync_copy(src.at[pl.ds(toff, tail)], vals0.at[pl.ds(0, tail)])
        pltpu.sync_copy(vals0.at[pl.ds(0, tail)], acc.at[idxt], add=True)

        plsc.subcore_barrier()
        pltpu.sync_copy(acc.at[pl.ds(rbase, NROW)],
                        out.at[pl.ds(c * N + rbase, NROW)])

        @pl.when(last)
        def _():
            pltpu.sync_copy(acc.at[pl.ds(tb, NTAIL)],
                            out.at[pl.ds(c * N + tb, NTAIL)])

    return body


@functools.cache
def _sc_gather(e):
    mesh = plsc.VectorSubcoreMesh(core_axis_name="c", subcore_axis_name="s")
    ept = e // NW
    return pl.kernel(
        _make_gather_body(ept),
        out_type=jax.ShapeDtypeStruct((e, LAT), jnp.float32),
        mesh=mesh,
        scratch_types=[
            pltpu.VMEM((ept,), jnp.int32),
            pltpu.VMEM((ept,), jnp.int32),
            pltpu.VMEM((GCH, LAT), jnp.float32),
            pltpu.VMEM((GCH, LAT), jnp.float32),
            pltpu.SemaphoreType.DMA,
            pltpu.SemaphoreType.DMA,
        ],
    )


@functools.cache
def _sc_scatter(e):
    mesh = plsc.VectorSubcoreMesh(core_axis_name="c", subcore_axis_name="s")
    ept = e // NW
    nch = ept // SCH
    tail = ept - nch * SCH
    return pl.kernel(
        _make_scatter_body(ept, nch, tail),
        out_type=jax.ShapeDtypeStruct((NC * N, LAT), jnp.float32),
        mesh=mesh,
        scratch_types=[
            pltpu.VMEM_SHARED((N, LAT), jnp.float32),
            pltpu.VMEM((SCH, LAT), jnp.float32),
            pltpu.VMEM((SCH, LAT), jnp.float32),
            pltpu.VMEM((SCH,), jnp.int32),
            pltpu.VMEM((SCH,), jnp.int32),
            pltpu.VMEM((tail,), jnp.int32),
            pltpu.SemaphoreType.DMA,
            pltpu.SemaphoreType.DMA,
        ],
    )


def _ln(h, g, be):
    mu = jnp.mean(h, axis=-1, keepdims=True)
    var = jnp.mean((h - mu) ** 2, axis=-1, keepdims=True)
    h = (h - mu) * lax.rsqrt(var + 1e-5)
    return h * g + be


def _dot(a, b):
    return jnp.dot(a, b, precision=lax.Precision.DEFAULT,
                   preferred_element_type=jnp.float32)


def _mlp_tail(h, w2, b2, w3, b3, w4, b4, g, be):
    h = jnp.maximum(h, 0.0)
    h = jnp.maximum(_dot(h, w2[...]) + b2[...], 0.0)
    h = jnp.maximum(_dot(h, w3[...]) + b3[...], 0.0)
    h = _dot(h, w4[...]) + b4[...]
    return _ln(h, g[...], be[...])


def _enc_body(x, w1, b1, w2, b2, w3, b3, w4, b4, g, be, o_ref):
    h = _dot(x[...], w1[...]) + b1[...]
    o_ref[...] = _mlp_tail(h, w2, b2, w3, b3, w4, b4, g, be)


def _proj_body(nl, wam, wbm, b1m, waw, wbw, b1w,
               psm_ref, prm_ref, psw_ref, prw_ref):
    x = nl[...]
    psm_ref[...] = _dot(x, wam[...]) + b1m[...]
    prm_ref[...] = _dot(x, wbm[...])
    psw_ref[...] = _dot(x, waw[...]) + b1w[...]
    prw_ref[...] = _dot(x, wbw[...])


def _edge_body(gsum, el, wc, w2, b2, w3, b3, w4, b4, g, be, up_ref):
    x = el[...]
    h = gsum[...] + _dot(x, wc[...])
    ne = _mlp_tail(h, w2, b2, w3, b3, w4, b4, g, be)
    up_ref[...] = x + ne


def _node_body(nl, pm0, pm1, qm0, qm1, pw0, pw1, qw0, qw1,
               wa, wb, wc, b1, w2, b2, w3, b3, w4, b4, g, be, o_ref):
    x = nl[...]
    am = pm0[...] + pm1[...] - qm0[...] - qm1[...]
    aw = pw0[...] + pw1[...] - qw0[...] - qw1[...]
    h = (_dot(x, wa[...]) + _dot(am, wb[...])
         + _dot(aw, wc[...]) + b1[...])
    o_ref[...] = x + _mlp_tail(h, w2, b2, w3, b3, w4, b4, g, be)


def _full(a):
    nd = a.ndim
    return pl.BlockSpec(a.shape, lambda i, _nd=nd: (0,) * _nd)


def _rows(blk, width):
    return pl.BlockSpec((blk, width), lambda i: (i, 0))


def _run_enc(x, p, blk):
    e, ind = x.shape
    ws = list(p["W"])
    bs = [b.reshape(1, LAT) for b in p["b"]]
    g = p["g"].reshape(1, LAT)
    be = p["be"].reshape(1, LAT)
    flat = [ws[0], bs[0], ws[1], bs[1], ws[2], bs[2], ws[3], bs[3], g, be]
    return pl.pallas_call(
        _enc_body,
        grid=(e // blk,),
        in_specs=[_rows(blk, ind)] + [_full(a) for a in flat],
        out_specs=_rows(blk, LAT),
        out_shape=jax.ShapeDtypeStruct((e, LAT), jnp.float32),
    )(x, *flat)


def _run_proj(nl, wm, ww, blk):
    flat = [wm["wa"], wm["wb"], wm["b1"], ww["wa"], ww["wb"], ww["b1"]]
    return pl.pallas_call(
        _proj_body,
        grid=(N // blk,),
        in_specs=[_rows(blk, LAT)] + [_full(a) for a in flat],
        out_specs=[_rows(blk, LAT)] * 4,
        out_shape=[jax.ShapeDtypeStruct((N, LAT), jnp.float32)] * 4,
    )(nl, *flat)


def _run_edge(gsum, el, w, blk):
    e = el.shape[0]
    flat = [w["wc"], w["w2"], w["b2"], w["w3"],
            w["b3"], w["w4"], w["b4"], w["g"], w["be"]]
    return pl.pallas_call(
        _edge_body,
        grid=(e // blk,),
        in_specs=[_rows(blk, LAT)] * 2 + [_full(a) for a in flat],
        out_specs=_rows(blk, LAT),
        out_shape=jax.ShapeDtypeStruct((e, LAT), jnp.float32),
    )(gsum, el, *flat)


def _run_node(nl, pm, qm, pw, qw, w, blk):
    flat = [w["wa"], w["wb"], w["wc"], w["b1"], w["w2"], w["b2"], w["w3"],
            w["b3"], w["w4"], w["b4"], w["g"], w["be"]]
    nb = N // blk
    part0 = pl.BlockSpec((blk, LAT), lambda i: (i, 0))
    part1 = pl.BlockSpec((blk, LAT), lambda i, _nb=nb: (i + _nb, 0))
    return pl.pallas_call(
        _node_body,
        grid=(nb,),
        in_specs=[_rows(blk, LAT), part0, part1, part0, part1,
                  part0, part1, part0, part1]
        + [_full(a) for a in flat],
        out_specs=_rows(blk, LAT),
        out_shape=jax.ShapeDtypeStruct((N, LAT), jnp.float32),
    )(nl, pm, pm, qm, qm, pw, pw, qw, qw, *flat)


def _stack_block(blocks, role):
    def cat(f):
        return jnp.stack([f(b[role]) for b in blocks])

    return {
        "wa": cat(lambda p: p["W"][0][:LAT]),
        "wb": cat(lambda p: p["W"][0][LAT:2 * LAT]),
        "wc": cat(lambda p: p["W"][0][2 * LAT:]),
        "b1": cat(lambda p: p["b"][0].reshape(1, LAT)),
        "w2": cat(lambda p: p["W"][1]),
        "b2": cat(lambda p: p["b"][1].reshape(1, LAT)),
        "w3": cat(lambda p: p["W"][2]),
        "b3": cat(lambda p: p["b"][2].reshape(1, LAT)),
        "w4": cat(lambda p: p["W"][3]),
        "b4": cat(lambda p: p["b"][3].reshape(1, LAT)),
        "g": cat(lambda p: p["g"].reshape(1, LAT)),
        "be": cat(lambda p: p["be"].reshape(1, LAT)),
    }


def kernel(node_features, mesh_features, world_features, params,
           mesh_senders, mesh_receivers, world_senders, world_receivers):
    ms = mesh_senders.astype(jnp.int32)
    mr = mesh_receivers.astype(jnp.int32)
    ws = world_senders.astype(jnp.int32)
    wr = world_receivers.astype(jnp.int32)

    enc = params["enc"]
    node_lat = _run_enc(node_features, enc["node"], 2000)
    mesh_lat = _run_enc(mesh_features, enc["mesh"], 8000)
    world_lat = _run_enc(world_features, enc["world"], 8000)

    blocks = params["blocks"]
    wts = {
        "mesh": _stack_block(blocks, "mesh"),
        "world": _stack_block(blocks, "world"),
        "node": _stack_block(blocks, "node"),
    }
    zz = jnp.zeros((NROW, LAT), jnp.float32)

    pm0 = _sc_scatter(EM)(mesh_lat, mr, zz)
    pw0 = _sc_scatter(EW)(world_lat, wr, zz)

    def step(carry, w):
        nl, ml, wl, qm, qw = carry
        psm, prm, psw, prw = _run_proj(nl, w["mesh"], w["world"], 2000)
        gm = _sc_gather(EM)(psm, prm, ms, mr)
        gw = _sc_gather(EW)(psw, prw, ws, wr)
        ml2 = _run_edge(gm, ml, w["mesh"], 16000)
        pm = _sc_scatter(EM)(ml2, mr, zz)
        wl2 = _run_edge(gw, wl, w["world"], 16000)
        pw = _sc_scatter(EW)(wl2, wr, zz)
        nl2 = _run_node(nl, pm, qm, pw, qw, w["node"], 2000)
        return (nl2, ml2, wl2, pm, pw), None

    (node_lat, mesh_lat, world_lat, _, _), _ = lax.scan(
        step, (node_lat, mesh_lat, world_lat, pm0, pw0), wts, unroll=18)
    return (node_lat, mesh_lat, world_lat)
